# Initial kernel scaffold; baseline (speedup 1.0000x reference)
import jax
import jax.numpy as jnp
from jax.experimental import pallas as pl


def kernel(emb, edge_index, d, scale):
    raise NotImplementedError("write your pallas kernel here")


# speedup vs baseline: 1.1108x; 1.1108x over previous
import functools

import jax
import jax.numpy as jnp
from jax import lax
from jax.experimental import pallas as pl
from jax.experimental.pallas import tpu as pltpu
from jax.experimental.pallas import tpu_sc as plsc

N_NODES = 10000
N_EDGES = 320000
D = 128

NC = 2
NS = 16
L = 16
NW = NC * NS
PER_W = N_EDGES // NW
CHUNK = 80
N_CHUNKS = PER_W // CHUNK
KU = 16


def _norm_body(emb_ref, d_ref, scale_ref, a_ref, b_ref):
    x = emb_ref[...]
    n = jnp.sqrt(jnp.sum(x * x, axis=1, keepdims=True))
    bn = x / jnp.maximum(n, 1e-12)
    b_ref[...] = bn
    a_ref[...] = bn * (d_ref[...] * scale_ref[...])


@jax.jit
def _normalize(emb, d2, s2):
    blk = 1000
    grid = N_NODES // blk
    return pl.pallas_call(
        _norm_body,
        grid=(grid,),
        in_specs=[
            pl.BlockSpec((blk, D), lambda i: (i, 0)),
            pl.BlockSpec((1, D), lambda i: (0, 0)),
            pl.BlockSpec((1, 1), lambda i: (0, 0)),
        ],
        out_specs=[
            pl.BlockSpec((blk, D), lambda i: (i, 0)),
            pl.BlockSpec((blk, D), lambda i: (i, 0)),
        ],
        out_shape=[
            jax.ShapeDtypeStruct((N_NODES, D), jnp.float32),
            jax.ShapeDtypeStruct((N_NODES, D), jnp.float32),
        ],
    )(emb, d2, s2)


def _edge_dot_body(a_hbm, b_hbm, src_hbm, dst_hbm, out_hbm,
                   sidx, didx, arows, brows, z, sem):
    wid = lax.axis_index("s") * NC + lax.axis_index("c")
    base = wid * PER_W

    pltpu.sync_copy(src_hbm.at[pl.ds(base, PER_W)], sidx)
    pltpu.sync_copy(dst_hbm.at[pl.ds(base, PER_W)], didx)

    def chunk_body(c, carry):
        off = c * CHUNK
        cp_a = pltpu.async_copy(a_hbm.at[sidx.at[pl.ds(off, CHUNK)]], arows, sem)
        cp_b = pltpu.async_copy(b_hbm.at[didx.at[pl.ds(off, CHUNK)]], brows, sem)
        cp_a.wait()
        cp_b.wait()
        for g in range(CHUNK // L):
            rows = jnp.full((L,), g * L, jnp.int32) + lax.iota(jnp.int32, L)

            def kbody(kk, acc):
                for u in range(KU):
                    kv = kk * KU + u
                    col = jnp.full((L,), kv, jnp.int32)
                    av = plsc.load_gather(arows, [rows, col])
                    bv = plsc.load_gather(brows, [rows, col])
                    acc = acc + av * bv
                return acc

            acc = lax.fori_loop(0, D // KU, kbody, jnp.zeros((L,), jnp.float32))
            z[pl.ds(off + g * L, L)] = acc
        return carry

    lax.fori_loop(0, N_CHUNKS, chunk_body, 0)
    pltpu.sync_copy(z, out_hbm.at[pl.ds(base, PER_W)])


@jax.jit
def _sc_edge_dot(a, b, src, dst):
    mesh = plsc.VectorSubcoreMesh(core_axis_name="c", subcore_axis_name="s")
    f = functools.partial(
        pl.kernel,
        mesh=mesh,
        compiler_params=pltpu.CompilerParams(needs_layout_passes=False),
        out_type=jax.ShapeDtypeStruct((N_EDGES,), jnp.float32),
        scratch_types=[
            pltpu.VMEM((PER_W,), jnp.int32),
            pltpu.VMEM((PER_W,), jnp.int32),
            pltpu.VMEM((CHUNK, D), jnp.float32),
            pltpu.VMEM((CHUNK, D), jnp.float32),
            pltpu.VMEM((PER_W,), jnp.float32),
            pltpu.SemaphoreType.DMA,
        ],
    )(_edge_dot_body)
    return f(a, b, src, dst)


def kernel(emb, edge_index, d, scale):
    d2 = d.reshape(1, D).astype(jnp.float32)
    s2 = scale.reshape(1, 1).astype(jnp.float32)
    a, b = _normalize(emb, d2, s2)
    src = edge_index[0].astype(jnp.int32)
    dst = edge_index[1].astype(jnp.int32)
    z = _sc_edge_dot(a, b, src, dst)
    return z.reshape(N_EDGES, 1)


# speedup vs baseline: 1.2750x; 1.1478x over previous
import functools

import jax
import jax.numpy as jnp
from jax import lax
from jax.experimental import pallas as pl
from jax.experimental.pallas import tpu as pltpu
from jax.experimental.pallas import tpu_sc as plsc

N_NODES = 10000
N_EDGES = 320000
D = 128

NC = 2
NS = 16
L = 16
NW = NC * NS
PER_W = N_EDGES // NW
CHUNK = 80
N_CHUNKS = PER_W // CHUNK
KU = 16


def _norm_body(emb_ref, d_ref, scale_ref, a_ref, b_ref):
    x = emb_ref[...]
    n = jnp.sqrt(jnp.sum(x * x, axis=1, keepdims=True))
    bn = x / jnp.maximum(n, 1e-12)
    b_ref[...] = bn
    a_ref[...] = bn * (d_ref[...] * scale_ref[...])


@jax.jit
def _normalize(emb, d2, s2):
    blk = 1000
    grid = N_NODES // blk
    return pl.pallas_call(
        _norm_body,
        grid=(grid,),
        in_specs=[
            pl.BlockSpec((blk, D), lambda i: (i, 0)),
            pl.BlockSpec((1, D), lambda i: (0, 0)),
            pl.BlockSpec((1, 1), lambda i: (0, 0)),
        ],
        out_specs=[
            pl.BlockSpec((blk, D), lambda i: (i, 0)),
            pl.BlockSpec((blk, D), lambda i: (i, 0)),
        ],
        out_shape=[
            jax.ShapeDtypeStruct((N_NODES, D), jnp.float32),
            jax.ShapeDtypeStruct((N_NODES, D), jnp.float32),
        ],
    )(emb, d2, s2)


def _edge_dot_body(a_hbm, b_hbm, src_hbm, dst_hbm, out_hbm,
                   sidx, didx, arows0, brows0, arows1, brows1, z,
                   sem0, sem1):
    wid = lax.axis_index("s") * NC + lax.axis_index("c")
    base = wid * PER_W

    pltpu.sync_copy(src_hbm.at[pl.ds(base, PER_W)], sidx)
    pltpu.sync_copy(dst_hbm.at[pl.ds(base, PER_W)], didx)

    bufs = ((arows0, brows0, sem0), (arows1, brows1, sem1))

    def start(c, p):
        ar, br, sem = bufs[p]
        off = c * CHUNK
        pltpu.async_copy(a_hbm.at[sidx.at[pl.ds(off, CHUNK)]], ar, sem)
        pltpu.async_copy(b_hbm.at[didx.at[pl.ds(off, CHUNK)]], br, sem)

    def finish(c, p):
        ar, br, sem = bufs[p]
        pltpu.make_async_copy(a_hbm.at[pl.ds(0, CHUNK)], ar, sem).wait()
        pltpu.make_async_copy(b_hbm.at[pl.ds(0, CHUNK)], br, sem).wait()
        for g in range(CHUNK // L):
            rows = jnp.full((L,), g * L, jnp.int32) + lax.iota(jnp.int32, L)

            def kbody(kk, carry):
                col, acc = carry
                for _ in range(KU):
                    av = plsc.load_gather(ar, [rows, col])
                    bv = plsc.load_gather(br, [rows, col])
                    acc = acc + av * bv
                    col = col + 1
                return col, acc

            _, acc = lax.fori_loop(
                0, D // KU, kbody,
                (jnp.zeros((L,), jnp.int32), jnp.zeros((L,), jnp.float32)))
            z[pl.ds(c * CHUNK + g * L, L)] = acc

    start(0, 0)

    def pair_body(i, carry):
        c = 2 * i
        start(c + 1, 1)
        finish(c, 0)
        start(c + 2, 0)
        finish(c + 1, 1)
        return carry

    lax.fori_loop(0, (N_CHUNKS - 1) // 2, pair_body, 0)
    finish(N_CHUNKS - 1, 0)
    pltpu.sync_copy(z, out_hbm.at[pl.ds(base, PER_W)])


@jax.jit
def _sc_edge_dot(a, b, src, dst):
    mesh = plsc.VectorSubcoreMesh(core_axis_name="c", subcore_axis_name="s")
    f = functools.partial(
        pl.kernel,
        mesh=mesh,
        compiler_params=pltpu.CompilerParams(needs_layout_passes=False),
        out_type=jax.ShapeDtypeStruct((N_EDGES,), jnp.float32),
        scratch_types=[
            pltpu.VMEM((PER_W,), jnp.int32),
            pltpu.VMEM((PER_W,), jnp.int32),
            pltpu.VMEM((CHUNK, D), jnp.float32),
            pltpu.VMEM((CHUNK, D), jnp.float32),
            pltpu.VMEM((CHUNK, D), jnp.float32),
            pltpu.VMEM((CHUNK, D), jnp.float32),
            pltpu.VMEM((PER_W,), jnp.float32),
            pltpu.SemaphoreType.DMA,
            pltpu.SemaphoreType.DMA,
        ],
    )(_edge_dot_body)
    return f(a, b, src, dst)


def kernel(emb, edge_index, d, scale):
    d2 = d.reshape(1, D).astype(jnp.float32)
    s2 = scale.reshape(1, 1).astype(jnp.float32)
    a, b = _normalize(emb, d2, s2)
    src = edge_index[0].astype(jnp.int32)
    dst = edge_index[1].astype(jnp.int32)
    z = _sc_edge_dot(a, b, src, dst)
    return z.reshape(N_EDGES, 1)


# speedup vs baseline: 7.8189x; 6.1327x over previous
import functools

import jax
import jax.numpy as jnp
from jax import lax
from jax.experimental import pallas as pl
from jax.experimental.pallas import tpu as pltpu
from jax.experimental.pallas import tpu_sc as plsc

N_NODES = 10000
N_EDGES = 320000
D = 128

NC = 2
NS = 16
L = 16
NW = NC * NS
PER_W = N_EDGES // NW
CHUNK = 80
N_CHUNKS = PER_W // CHUNK
KU = 16


def _norm_body(emb_ref, d_ref, scale_ref, a_ref, b_ref):
    x = emb_ref[...]
    n = jnp.sqrt(jnp.sum(x * x, axis=1, keepdims=True))
    bn = x / jnp.maximum(n, 1e-12)
    b_ref[...] = bn
    a_ref[...] = bn * (d_ref[...] * scale_ref[...])


@jax.jit
def _normalize(emb, d2, s2):
    blk = 1000
    grid = N_NODES // blk
    return pl.pallas_call(
        _norm_body,
        grid=(grid,),
        in_specs=[
            pl.BlockSpec((blk, D), lambda i: (i, 0)),
            pl.BlockSpec((1, D), lambda i: (0, 0)),
            pl.BlockSpec((1, 1), lambda i: (0, 0)),
        ],
        out_specs=[
            pl.BlockSpec((blk, D), lambda i: (i, 0)),
            pl.BlockSpec((blk, D), lambda i: (i, 0)),
        ],
        out_shape=[
            jax.ShapeDtypeStruct((N_NODES, D), jnp.float32),
            jax.ShapeDtypeStruct((N_NODES, D), jnp.float32),
        ],
    )(emb, d2, s2)


def _edge_dot_body(a_hbm, b_hbm, src_hbm, dst_hbm, out_hbm,
                   sidx, didx, arows0, brows0, arows1, brows1, z,
                   sem0, sem1):
    wid = lax.axis_index("s") * NC + lax.axis_index("c")
    base = wid * PER_W

    pltpu.sync_copy(src_hbm.at[pl.ds(base, PER_W)], sidx)
    pltpu.sync_copy(dst_hbm.at[pl.ds(base, PER_W)], didx)

    bufs = ((arows0, brows0, sem0), (arows1, brows1, sem1))

    def start(c, p):
        ar, br, sem = bufs[p]
        off = c * CHUNK
        pltpu.async_copy(a_hbm.at[sidx.at[pl.ds(off, CHUNK)]], ar, sem)
        pltpu.async_copy(b_hbm.at[didx.at[pl.ds(off, CHUNK)]], br, sem)

    def finish(c, p):
        ar, br, sem = bufs[p]
        pltpu.make_async_copy(a_hbm.at[pl.ds(0, CHUNK)], ar, sem).wait()
        pltpu.make_async_copy(b_hbm.at[pl.ds(0, CHUNK)], br, sem).wait()
        lane = lax.iota(jnp.int32, L)
        for g in range(CHUNK // L):
            rows = jnp.full((L,), g * L, jnp.int32) + lane

            def kbody(kk, carry):
                col, acc = carry
                for _ in range(KU):
                    av = plsc.load_gather(ar, [rows, col])
                    bv = plsc.load_gather(br, [rows, col])
                    acc = acc + av * bv
                    col = (col + 1) & (D - 1)
                return col, acc

            _, acc = lax.fori_loop(
                0, D // KU, kbody,
                (lane, jnp.zeros((L,), jnp.float32)))
            z[pl.ds(c * CHUNK + g * L, L)] = acc

    start(0, 0)

    def pair_body(i, carry):
        c = 2 * i
        start(c + 1, 1)
        finish(c, 0)
        start(c + 2, 0)
        finish(c + 1, 1)
        return carry

    lax.fori_loop(0, (N_CHUNKS - 1) // 2, pair_body, 0)
    finish(N_CHUNKS - 1, 0)
    pltpu.sync_copy(z, out_hbm.at[pl.ds(base, PER_W)])


@jax.jit
def _sc_edge_dot(a, b, src, dst):
    mesh = plsc.VectorSubcoreMesh(core_axis_name="c", subcore_axis_name="s")
    f = functools.partial(
        pl.kernel,
        mesh=mesh,
        compiler_params=pltpu.CompilerParams(needs_layout_passes=False),
        out_type=jax.ShapeDtypeStruct((N_EDGES,), jnp.float32),
        scratch_types=[
            pltpu.VMEM((PER_W,), jnp.int32),
            pltpu.VMEM((PER_W,), jnp.int32),
            pltpu.VMEM((CHUNK, D), jnp.float32),
            pltpu.VMEM((CHUNK, D), jnp.float32),
            pltpu.VMEM((CHUNK, D), jnp.float32),
            pltpu.VMEM((CHUNK, D), jnp.float32),
            pltpu.VMEM((PER_W,), jnp.float32),
            pltpu.SemaphoreType.DMA,
            pltpu.SemaphoreType.DMA,
        ],
    )(_edge_dot_body)
    return f(a, b, src, dst)


def kernel(emb, edge_index, d, scale):
    d2 = d.reshape(1, D).astype(jnp.float32)
    s2 = scale.reshape(1, 1).astype(jnp.float32)
    a, b = _normalize(emb, d2, s2)
    src = edge_index[0].astype(jnp.int32)
    dst = edge_index[1].astype(jnp.int32)
    z = _sc_edge_dot(a, b, src, dst)
    return z.reshape(N_EDGES, 1)
